# baseline (speedup 1.0000x reference)
import functools

import jax
import jax.numpy as jnp
from jax import lax
from jax.experimental import pallas as pl
from jax.experimental.pallas import tpu as pltpu
from jax.experimental.pallas import tpu_sc as plsc

NL = 10000
NCL = 10000
NE = 160000
D = 256
DFF = 1024
NLAYER = 3

NSC = 2
NTEC = 16
NLP = 10240
EW = NE // NTEC
B = 128
EWP = 10240
NB = EWP // B
PADROW = NL + 16
DC = 32
NCH = D // DC
NR = NCH // NSC
RS = NLP // NTEC
SB = 128
NSB = RS // SB
EPS = 1e-6

@functools.cache
def _mesh():
    return plsc.VectorSubcoreMesh(
        core_axis_name="c", subcore_axis_name="s", num_cores=NSC,
        num_subcores=NTEC)


def _fill(ref, rows, cols, val):
    vec = jnp.full((16,), val, jnp.float32)

    def body(i, carry):
        for q in range(cols // 16):
            ref[i, pl.ds(q * 16, 16)] = vec
        return carry

    lax.fori_loop(0, rows, body, 0)


def _z_body(p_cls, p_lit, n_cls, n_lit, wb, zinv_out,
            zt_p, zt_n, zw, pcv, plv, ncv, nlv, gz, ga, gb, buf_a, buf_b,
            wvm, sem0, sem1, sem2, sem3):
    c = lax.axis_index("c")
    s = lax.axis_index("s")
    row0 = s * RS
    pltpu.sync_copy(p_cls.at[s], pcv)
    pltpu.sync_copy(p_lit.at[s], plv)
    pltpu.sync_copy(n_cls.at[s], ncv)
    pltpu.sync_copy(n_lit.at[s], nlv)
    pltpu.sync_copy(wb, wvm)
    _fill(buf_a, RS, 16, 0.0)
    pltpu.sync_copy(buf_a, zt_p.at[pl.ds(row0, RS)])
    pltpu.sync_copy(buf_a, zt_n.at[pl.ds(row0, RS)])
    pltpu.sync_copy(buf_a, zw.at[pl.ds(row0, RS)])
    _fill(gz, B, 16, 1.0)
    plsc.subcore_barrier()

    def deg(idx_v, acc_sp):
        def body(t, carry):
            d0 = pltpu.async_copy(gz, acc_sp.at[idx_v.at[2 * t]], sem0,
                                  add=True)
            d1 = pltpu.async_copy(gz, acc_sp.at[idx_v.at[2 * t + 1]], sem1,
                                  add=True)
            d0.wait()
            d1.wait()
            return carry

        lax.fori_loop(0, NB // 2, body, 0)

    deg(pcv, zt_p)
    deg(ncv, zt_n)
    plsc.subcore_barrier()
    pltpu.sync_copy(zt_p.at[pl.ds(row0, RS)], buf_a)
    pltpu.sync_copy(zt_n.at[pl.ds(row0, RS)], buf_b)
    w0 = wvm[0]
    w1 = wvm[1]
    w2 = wvm[2]
    w3 = wvm[3]

    def comb(i, carry):
        a = buf_a[i]
        b = buf_b[i]
        buf_a[i] = w0 * a + w1 * b
        buf_b[i] = w2 * a + w3 * b
        return carry

    lax.fori_loop(0, RS, comb, 0)
    pltpu.sync_copy(buf_a, zt_p.at[pl.ds(row0, RS)])
    pltpu.sync_copy(buf_b, zt_n.at[pl.ds(row0, RS)])
    plsc.subcore_barrier()

    def lit_scatter(cls_v, lit_v, src_sp):
        def body(t, carry):
            j0 = 2 * t
            j1 = 2 * t + 1
            dg0 = pltpu.async_copy(src_sp.at[cls_v.at[j0]], ga, sem0)
            dg1 = pltpu.async_copy(src_sp.at[cls_v.at[j1]], gb, sem1)
            dg0.wait()
            ds0 = pltpu.async_copy(ga, zw.at[lit_v.at[j0]], sem2, add=True)
            dg1.wait()
            ds1 = pltpu.async_copy(gb, zw.at[lit_v.at[j1]], sem3, add=True)
            ds0.wait()
            ds1.wait()
            return carry

        lax.fori_loop(0, NB // 2, body, 0)

    lit_scatter(pcv, plv, zt_p)
    lit_scatter(ncv, nlv, zt_n)
    plsc.subcore_barrier()
    pltpu.sync_copy(zw.at[pl.ds(row0, RS)], buf_a)

    def recip(i, carry):
        buf_a[i] = 1.0 / (buf_a[i] + EPS)
        return carry

    lax.fori_loop(0, RS, recip, 0)

    @pl.when(c == 0)
    def _():
        pltpu.sync_copy(buf_a, zinv_out.at[pl.ds(row0, RS)])


@functools.cache
def _z_kernel():
    return pl.kernel(
        _z_body,
        out_type=jax.ShapeDtypeStruct((NLP, 16), jnp.float32),
        mesh=_mesh(),
        compiler_params=pltpu.CompilerParams(use_tc_tiling_on_sc=False),
        scratch_types=[
            pltpu.VMEM_SHARED((NLP, 16), jnp.float32),
            pltpu.VMEM_SHARED((NLP, 16), jnp.float32),
            pltpu.VMEM_SHARED((NLP, 16), jnp.float32),
            pltpu.VMEM((NB, B), jnp.int32),
            pltpu.VMEM((NB, B), jnp.int32),
            pltpu.VMEM((NB, B), jnp.int32),
            pltpu.VMEM((NB, B), jnp.int32),
            pltpu.VMEM((B, 16), jnp.float32),
            pltpu.VMEM((B, 16), jnp.float32),
            pltpu.VMEM((B, 16), jnp.float32),
            pltpu.VMEM((RS, 16), jnp.float32),
            pltpu.VMEM((RS, 16), jnp.float32),
            pltpu.VMEM((4, 16), jnp.float32),
            pltpu.SemaphoreType.DMA,
            pltpu.SemaphoreType.DMA,
            pltpu.SemaphoreType.DMA,
            pltpu.SemaphoreType.DMA,
        ],
    )


def _mp_body(nh, p_cls, p_lit, n_cls, n_lit, wb, o_out,
             wv_sp, tp_sp, tn_sp, pcv, plv, ncv, nlv, ga, gb, buf_a, buf_b,
             wvm, sem0, sem1, sem2, sem3):
    c = lax.axis_index("c")
    s = lax.axis_index("s")
    row0 = s * RS
    pltpu.sync_copy(p_cls.at[s], pcv)
    pltpu.sync_copy(p_lit.at[s], plv)
    pltpu.sync_copy(n_cls.at[s], ncv)
    pltpu.sync_copy(n_lit.at[s], nlv)
    pltpu.sync_copy(wb, wvm)
    w0 = wvm[0]
    w1 = wvm[1]
    w2 = wvm[2]
    w3 = wvm[3]
    def edge_pass(gat_v, sct_v, src, dst_sp):
        def body(t, carry):
            j0 = 2 * t
            j1 = 2 * t + 1
            dg0 = pltpu.async_copy(src.at[gat_v.at[j0]], ga, sem0)
            dg1 = pltpu.async_copy(src.at[gat_v.at[j1]], gb, sem1)
            dg0.wait()
            ds0 = pltpu.async_copy(ga, dst_sp.at[sct_v.at[j0]], sem2,
                                   add=True)
            dg1.wait()
            ds1 = pltpu.async_copy(gb, dst_sp.at[sct_v.at[j1]], sem3,
                                   add=True)
            ds0.wait()
            ds1.wait()
            return carry

        lax.fori_loop(0, NB // 2, body, 0)

    for r in range(NR):
        ch = NSC * r + c
        _fill(buf_a, SB, DC, 0.0)
        for k in range(NSB):
            pltpu.sync_copy(buf_a, tp_sp.at[pl.ds(row0 + k * SB, SB)])
            pltpu.sync_copy(buf_a, tn_sp.at[pl.ds(row0 + k * SB, SB)])
            pltpu.sync_copy(buf_a, wv_sp.at[pl.ds(row0 + k * SB, SB)])
        plsc.subcore_barrier()

        edge_pass(plv, pcv, nh.at[ch], tp_sp)
        edge_pass(nlv, ncv, nh.at[ch], tn_sp)
        plsc.subcore_barrier()
        for k in range(NSB):
            base = row0 + k * SB
            pltpu.sync_copy(tp_sp.at[pl.ds(base, SB)], buf_a)
            pltpu.sync_copy(tn_sp.at[pl.ds(base, SB)], buf_b)

            def comb(i, carry):
                for q in range(DC // 16):
                    sl = pl.ds(q * 16, 16)
                    a = buf_a[i, sl]
                    b = buf_b[i, sl]
                    buf_a[i, sl] = w0 * a + w1 * b
                    buf_b[i, sl] = w2 * a + w3 * b
                return carry

            lax.fori_loop(0, SB, comb, 0)
            pltpu.sync_copy(buf_a, tp_sp.at[pl.ds(base, SB)])
            pltpu.sync_copy(buf_b, tn_sp.at[pl.ds(base, SB)])
        plsc.subcore_barrier()
        edge_pass(pcv, plv, tp_sp, wv_sp)
        edge_pass(ncv, nlv, tn_sp, wv_sp)
        plsc.subcore_barrier()
        pltpu.sync_copy(wv_sp.at[pl.ds(row0, RS)],
                        o_out.at[ch, pl.ds(row0, RS)])


@functools.cache
def _mp_kernel():
    return pl.kernel(
        _mp_body,
        out_type=jax.ShapeDtypeStruct((NCH, NLP, DC), jnp.float32),
        mesh=_mesh(),
        compiler_params=pltpu.CompilerParams(use_tc_tiling_on_sc=False),
        scratch_types=[
            pltpu.VMEM_SHARED((NLP, DC), jnp.float32),
            pltpu.VMEM_SHARED((NLP, DC), jnp.float32),
            pltpu.VMEM_SHARED((NLP, DC), jnp.float32),
            pltpu.VMEM((NB, B), jnp.int32),
            pltpu.VMEM((NB, B), jnp.int32),
            pltpu.VMEM((NB, B), jnp.int32),
            pltpu.VMEM((NB, B), jnp.int32),
            pltpu.VMEM((B, DC), jnp.float32),
            pltpu.VMEM((B, DC), jnp.float32),
            pltpu.VMEM((SB, DC), jnp.float32),
            pltpu.VMEM((SB, DC), jnp.float32),
            pltpu.VMEM((4, 16), jnp.float32),
            pltpu.SemaphoreType.DMA,
            pltpu.SemaphoreType.DMA,
            pltpu.SemaphoreType.DMA,
            pltpu.SemaphoreType.DMA,
        ],
    )


def _ln_block(x_ref, g_ref, b_ref, o_ref):
    x = x_ref[...]
    mu = jnp.mean(x, axis=-1, keepdims=True)
    var = jnp.mean((x - mu) ** 2, axis=-1, keepdims=True)
    o_ref[...] = (x - mu) / jnp.sqrt(var + EPS) * g_ref[...] + b_ref[...]


_TC_R = 1000


def _tc_ln(x, g, b):
    return pl.pallas_call(
        _ln_block,
        grid=(NL // _TC_R,),
        in_specs=[
            pl.BlockSpec((_TC_R, D), lambda i: (i, 0)),
            pl.BlockSpec((1, D), lambda i: (0, 0)),
            pl.BlockSpec((1, D), lambda i: (0, 0)),
        ],
        out_specs=pl.BlockSpec((_TC_R, D), lambda i: (i, 0)),
        out_shape=jax.ShapeDtypeStruct((NL, D), jnp.float32),
    )(x, g, b)


def _tail_block(h_ref, o_ref, zi_ref, ga_ref, ba_ref, gb_ref, bb_ref,
                w1_ref, b1_ref, w2_ref, b2_ref, hn_ref, ln_ref):
    h2 = h_ref[...] + o_ref[...] * zi_ref[...]
    mu = jnp.mean(h2, axis=-1, keepdims=True)
    var = jnp.mean((h2 - mu) ** 2, axis=-1, keepdims=True)
    n2 = (h2 - mu) / jnp.sqrt(var + EPS) * ga_ref[...] + ba_ref[...]
    ff = jnp.maximum(jnp.dot(n2, w1_ref[...]) + b1_ref[...], 0.0)
    h3 = h2 + jnp.dot(ff, w2_ref[...]) + b2_ref[...]
    hn_ref[...] = h3
    mu2 = jnp.mean(h3, axis=-1, keepdims=True)
    var2 = jnp.mean((h3 - mu2) ** 2, axis=-1, keepdims=True)
    ln_ref[...] = (h3 - mu2) / jnp.sqrt(var2 + EPS) * gb_ref[...] + bb_ref[...]


def _tc_tail(h, o, zi, ga, ba, gb, bb, w1, b1, w2, b2):
    row = pl.BlockSpec((_TC_R, D), lambda i: (i, 0))
    vec = pl.BlockSpec((1, D), lambda i: (0, 0))
    return pl.pallas_call(
        _tail_block,
        grid=(NL // _TC_R,),
        in_specs=[
            row, row,
            pl.BlockSpec((_TC_R, 1), lambda i: (i, 0)),
            vec, vec, vec, vec,
            pl.BlockSpec((D, DFF), lambda i: (0, 0)),
            pl.BlockSpec((1, DFF), lambda i: (0, 0)),
            pl.BlockSpec((DFF, D), lambda i: (0, 0)),
            vec,
        ],
        out_specs=[row, row],
        out_shape=[
            jax.ShapeDtypeStruct((NL, D), jnp.float32),
            jax.ShapeDtypeStruct((NL, D), jnp.float32),
        ],
    )(h, o, zi, ga, ba, gb, bb, w1, b1, w2, b2)


def _edge_layout(idx_row):
    per_tec = idx_row.reshape(NTEC, EW)
    padded = jnp.pad(per_tec, ((0, 0), (0, EWP - EW)),
                     constant_values=PADROW)
    return padded.reshape(NTEC, NB, B)


def kernel(x, adj_pos, adj_neg, literals_weights, W1, b1, W2, b2, gamma,
           beta):
    p_cls = _edge_layout(adj_pos[0])
    p_lit = _edge_layout(adj_pos[1])
    n_cls = _edge_layout(adj_neg[0])
    n_lit = _edge_layout(adj_neg[1])
    wb = jnp.broadcast_to(literals_weights[:, None], (4, 16))
    zinv = _z_kernel()(p_cls, p_lit, n_cls, n_lit, wb)
    zcol = zinv[:NL, 0:1]
    h = x
    nh = _tc_ln(x, gamma[0:1], beta[0:1])
    for i in range(NLAYER):
        nh_p = jnp.pad(nh, ((0, NLP - NL), (0, 0)))
        nh_c = nh_p.reshape(NLP, NCH, DC).transpose(1, 0, 2)
        o_c = _mp_kernel()(nh_c, p_cls, p_lit, n_cls, n_lit, wb)
        o = o_c.transpose(1, 0, 2).reshape(NLP, D)[:NL]
        h, nh = _tc_tail(h, o, zcol, gamma[2 * i + 1:2 * i + 2],
                         beta[2 * i + 1:2 * i + 2],
                         gamma[2 * i + 2:2 * i + 3],
                         beta[2 * i + 2:2 * i + 3],
                         W1[i], b1[i:i + 1], W2[i], b2[i:i + 1])
    return nh


# speedup vs baseline: 1.2538x; 1.2538x over previous
import functools

import jax
import jax.numpy as jnp
from jax import lax
from jax.experimental import pallas as pl
from jax.experimental.pallas import tpu as pltpu
from jax.experimental.pallas import tpu_sc as plsc

NL = 10000
NCL = 10000
NE = 160000
D = 256
DFF = 1024
NLAYER = 3

NSC = 2
NTEC = 16
NLP = 10240
EW = NE // NTEC
B = 256
EWP = 10240
NB = EWP // B
PADROW = NL + 16
DC = 32
NCH = D // DC
NR = NCH // NSC
RS = NLP // NTEC
SB = 128
NSB = RS // SB
EPS = 1e-6

@functools.cache
def _mesh():
    return plsc.VectorSubcoreMesh(
        core_axis_name="c", subcore_axis_name="s", num_cores=NSC,
        num_subcores=NTEC)


def _fill(ref, rows, cols, val):
    vec = jnp.full((16,), val, jnp.float32)

    def body(i, carry):
        for q in range(cols // 16):
            ref[i, pl.ds(q * 16, 16)] = vec
        return carry

    lax.fori_loop(0, rows, body, 0)


def _z_body(p_cls, p_lit, n_cls, n_lit, wb, zinv_out,
            zt_p, zt_n, zw, pcv, plv, ncv, nlv, gz, ga, gb, buf_a, buf_b,
            wvm, sem0, sem1, sem2, sem3):
    c = lax.axis_index("c")
    s = lax.axis_index("s")
    row0 = s * RS
    pltpu.sync_copy(p_cls.at[s], pcv)
    pltpu.sync_copy(p_lit.at[s], plv)
    pltpu.sync_copy(n_cls.at[s], ncv)
    pltpu.sync_copy(n_lit.at[s], nlv)
    pltpu.sync_copy(wb, wvm)
    _fill(buf_a, RS, 16, 0.0)
    pltpu.sync_copy(buf_a, zt_p.at[pl.ds(row0, RS)])
    pltpu.sync_copy(buf_a, zt_n.at[pl.ds(row0, RS)])
    pltpu.sync_copy(buf_a, zw.at[pl.ds(row0, RS)])
    _fill(gz, B, 16, 1.0)
    plsc.subcore_barrier()

    def deg(idx_v, acc_sp):
        def body(t, carry):
            d0 = pltpu.async_copy(gz, acc_sp.at[idx_v.at[2 * t]], sem0,
                                  add=True)
            d1 = pltpu.async_copy(gz, acc_sp.at[idx_v.at[2 * t + 1]], sem1,
                                  add=True)
            d0.wait()
            d1.wait()
            return carry

        lax.fori_loop(0, NB // 2, body, 0)

    deg(pcv, zt_p)
    deg(ncv, zt_n)
    plsc.subcore_barrier()
    pltpu.sync_copy(zt_p.at[pl.ds(row0, RS)], buf_a)
    pltpu.sync_copy(zt_n.at[pl.ds(row0, RS)], buf_b)
    w0 = wvm[0]
    w1 = wvm[1]
    w2 = wvm[2]
    w3 = wvm[3]

    def comb(i, carry):
        a = buf_a[i]
        b = buf_b[i]
        buf_a[i] = w0 * a + w1 * b
        buf_b[i] = w2 * a + w3 * b
        return carry

    lax.fori_loop(0, RS, comb, 0)
    pltpu.sync_copy(buf_a, zt_p.at[pl.ds(row0, RS)])
    pltpu.sync_copy(buf_b, zt_n.at[pl.ds(row0, RS)])
    plsc.subcore_barrier()

    def lit_scatter(cls_v, lit_v, src_sp):
        def body(t, carry):
            j0 = 2 * t
            j1 = 2 * t + 1
            dg0 = pltpu.async_copy(src_sp.at[cls_v.at[j0]], ga, sem0)
            dg1 = pltpu.async_copy(src_sp.at[cls_v.at[j1]], gb, sem1)
            dg0.wait()
            ds0 = pltpu.async_copy(ga, zw.at[lit_v.at[j0]], sem2, add=True)
            dg1.wait()
            ds1 = pltpu.async_copy(gb, zw.at[lit_v.at[j1]], sem3, add=True)
            ds0.wait()
            ds1.wait()
            return carry

        lax.fori_loop(0, NB // 2, body, 0)

    lit_scatter(pcv, plv, zt_p)
    lit_scatter(ncv, nlv, zt_n)
    plsc.subcore_barrier()
    pltpu.sync_copy(zw.at[pl.ds(row0, RS)], buf_a)

    def recip(i, carry):
        buf_a[i] = 1.0 / (buf_a[i] + EPS)
        return carry

    lax.fori_loop(0, RS, recip, 0)

    @pl.when(c == 0)
    def _():
        pltpu.sync_copy(buf_a, zinv_out.at[pl.ds(row0, RS)])


@functools.cache
def _z_kernel():
    return pl.kernel(
        _z_body,
        out_type=jax.ShapeDtypeStruct((NLP, 16), jnp.float32),
        mesh=_mesh(),
        compiler_params=pltpu.CompilerParams(use_tc_tiling_on_sc=False),
        scratch_types=[
            pltpu.VMEM_SHARED((NLP, 16), jnp.float32),
            pltpu.VMEM_SHARED((NLP, 16), jnp.float32),
            pltpu.VMEM_SHARED((NLP, 16), jnp.float32),
            pltpu.VMEM((NB, B), jnp.int32),
            pltpu.VMEM((NB, B), jnp.int32),
            pltpu.VMEM((NB, B), jnp.int32),
            pltpu.VMEM((NB, B), jnp.int32),
            pltpu.VMEM((B, 16), jnp.float32),
            pltpu.VMEM((B, 16), jnp.float32),
            pltpu.VMEM((B, 16), jnp.float32),
            pltpu.VMEM((RS, 16), jnp.float32),
            pltpu.VMEM((RS, 16), jnp.float32),
            pltpu.VMEM((4, 16), jnp.float32),
            pltpu.SemaphoreType.DMA,
            pltpu.SemaphoreType.DMA,
            pltpu.SemaphoreType.DMA,
            pltpu.SemaphoreType.DMA,
        ],
    )


def _mp_body(nh, p_cls, p_lit, n_cls, n_lit, wb, o_out,
             wv_sp, tp_sp, tn_sp, pcv, plv, ncv, nlv, ga, gb, buf_a, buf_b,
             wvm, sem0, sem1, sem2, sem3):
    c = lax.axis_index("c")
    s = lax.axis_index("s")
    row0 = s * RS
    pltpu.sync_copy(p_cls.at[s], pcv)
    pltpu.sync_copy(p_lit.at[s], plv)
    pltpu.sync_copy(n_cls.at[s], ncv)
    pltpu.sync_copy(n_lit.at[s], nlv)
    pltpu.sync_copy(wb, wvm)
    w0 = wvm[0]
    w1 = wvm[1]
    w2 = wvm[2]
    w3 = wvm[3]
    def edge_pass(gat_v, sct_v, src, dst_sp):
        def body(t, carry):
            j0 = 2 * t
            j1 = 2 * t + 1
            dg0 = pltpu.async_copy(src.at[gat_v.at[j0]], ga, sem0)
            dg1 = pltpu.async_copy(src.at[gat_v.at[j1]], gb, sem1)
            dg0.wait()
            ds0 = pltpu.async_copy(ga, dst_sp.at[sct_v.at[j0]], sem2,
                                   add=True)
            dg1.wait()
            ds1 = pltpu.async_copy(gb, dst_sp.at[sct_v.at[j1]], sem3,
                                   add=True)
            ds0.wait()
            ds1.wait()
            return carry

        lax.fori_loop(0, NB // 2, body, 0)

    for r in range(NR):
        ch = NSC * r + c
        pltpu.sync_copy(nh.at[ch, pl.ds(row0, RS)],
                        wv_sp.at[pl.ds(row0, RS)])
        _fill(buf_a, SB, DC, 0.0)
        for k in range(NSB):
            pltpu.sync_copy(buf_a, tp_sp.at[pl.ds(row0 + k * SB, SB)])
            pltpu.sync_copy(buf_a, tn_sp.at[pl.ds(row0 + k * SB, SB)])
        plsc.subcore_barrier()

        edge_pass(plv, pcv, wv_sp, tp_sp)
        edge_pass(nlv, ncv, wv_sp, tn_sp)
        plsc.subcore_barrier()
        for k in range(NSB):
            base = row0 + k * SB
            pltpu.sync_copy(tp_sp.at[pl.ds(base, SB)], buf_a)
            pltpu.sync_copy(tn_sp.at[pl.ds(base, SB)], buf_b)

            def comb(i, carry):
                for q in range(DC // 16):
                    sl = pl.ds(q * 16, 16)
                    a = buf_a[i, sl]
                    b = buf_b[i, sl]
                    buf_a[i, sl] = w0 * a + w1 * b
                    buf_b[i, sl] = w2 * a + w3 * b
                return carry

            lax.fori_loop(0, SB, comb, 0)
            pltpu.sync_copy(buf_a, tp_sp.at[pl.ds(base, SB)])
            pltpu.sync_copy(buf_b, tn_sp.at[pl.ds(base, SB)])
        _fill(buf_a, SB, DC, 0.0)
        for k in range(NSB):
            pltpu.sync_copy(buf_a, wv_sp.at[pl.ds(row0 + k * SB, SB)])
        plsc.subcore_barrier()
        edge_pass(pcv, plv, tp_sp, wv_sp)
        edge_pass(ncv, nlv, tn_sp, wv_sp)
        plsc.subcore_barrier()
        pltpu.sync_copy(wv_sp.at[pl.ds(row0, RS)],
                        o_out.at[ch, pl.ds(row0, RS)])


@functools.cache
def _mp_kernel():
    return pl.kernel(
        _mp_body,
        out_type=jax.ShapeDtypeStruct((NCH, NLP, DC), jnp.float32),
        mesh=_mesh(),
        compiler_params=pltpu.CompilerParams(use_tc_tiling_on_sc=False),
        scratch_types=[
            pltpu.VMEM_SHARED((NLP, DC), jnp.float32),
            pltpu.VMEM_SHARED((NLP, DC), jnp.float32),
            pltpu.VMEM_SHARED((NLP, DC), jnp.float32),
            pltpu.VMEM((NB, B), jnp.int32),
            pltpu.VMEM((NB, B), jnp.int32),
            pltpu.VMEM((NB, B), jnp.int32),
            pltpu.VMEM((NB, B), jnp.int32),
            pltpu.VMEM((B, DC), jnp.float32),
            pltpu.VMEM((B, DC), jnp.float32),
            pltpu.VMEM((SB, DC), jnp.float32),
            pltpu.VMEM((SB, DC), jnp.float32),
            pltpu.VMEM((4, 16), jnp.float32),
            pltpu.SemaphoreType.DMA,
            pltpu.SemaphoreType.DMA,
            pltpu.SemaphoreType.DMA,
            pltpu.SemaphoreType.DMA,
        ],
    )


def _ln_block(x_ref, g_ref, b_ref, o_ref):
    x = x_ref[...]
    mu = jnp.mean(x, axis=-1, keepdims=True)
    var = jnp.mean((x - mu) ** 2, axis=-1, keepdims=True)
    o_ref[...] = (x - mu) / jnp.sqrt(var + EPS) * g_ref[...] + b_ref[...]


_TC_R = 1000


def _tc_ln(x, g, b):
    return pl.pallas_call(
        _ln_block,
        grid=(NL // _TC_R,),
        in_specs=[
            pl.BlockSpec((_TC_R, D), lambda i: (i, 0)),
            pl.BlockSpec((1, D), lambda i: (0, 0)),
            pl.BlockSpec((1, D), lambda i: (0, 0)),
        ],
        out_specs=pl.BlockSpec((_TC_R, D), lambda i: (i, 0)),
        out_shape=jax.ShapeDtypeStruct((NL, D), jnp.float32),
    )(x, g, b)


def _tail_block(h_ref, o_ref, zi_ref, ga_ref, ba_ref, gb_ref, bb_ref,
                w1_ref, b1_ref, w2_ref, b2_ref, hn_ref, ln_ref):
    h2 = h_ref[...] + o_ref[...] * zi_ref[...]
    mu = jnp.mean(h2, axis=-1, keepdims=True)
    var = jnp.mean((h2 - mu) ** 2, axis=-1, keepdims=True)
    n2 = (h2 - mu) / jnp.sqrt(var + EPS) * ga_ref[...] + ba_ref[...]
    ff = jnp.maximum(jnp.dot(n2, w1_ref[...]) + b1_ref[...], 0.0)
    h3 = h2 + jnp.dot(ff, w2_ref[...]) + b2_ref[...]
    hn_ref[...] = h3
    mu2 = jnp.mean(h3, axis=-1, keepdims=True)
    var2 = jnp.mean((h3 - mu2) ** 2, axis=-1, keepdims=True)
    ln_ref[...] = (h3 - mu2) / jnp.sqrt(var2 + EPS) * gb_ref[...] + bb_ref[...]


def _tc_tail(h, o, zi, ga, ba, gb, bb, w1, b1, w2, b2):
    row = pl.BlockSpec((_TC_R, D), lambda i: (i, 0))
    vec = pl.BlockSpec((1, D), lambda i: (0, 0))
    return pl.pallas_call(
        _tail_block,
        grid=(NL // _TC_R,),
        in_specs=[
            row, row,
            pl.BlockSpec((_TC_R, 1), lambda i: (i, 0)),
            vec, vec, vec, vec,
            pl.BlockSpec((D, DFF), lambda i: (0, 0)),
            pl.BlockSpec((1, DFF), lambda i: (0, 0)),
            pl.BlockSpec((DFF, D), lambda i: (0, 0)),
            vec,
        ],
        out_specs=[row, row],
        out_shape=[
            jax.ShapeDtypeStruct((NL, D), jnp.float32),
            jax.ShapeDtypeStruct((NL, D), jnp.float32),
        ],
    )(h, o, zi, ga, ba, gb, bb, w1, b1, w2, b2)


def _edge_layout(idx_row):
    per_tec = idx_row.reshape(NTEC, EW)
    padded = jnp.pad(per_tec, ((0, 0), (0, EWP - EW)),
                     constant_values=PADROW)
    return padded.reshape(NTEC, NB, B)


def kernel(x, adj_pos, adj_neg, literals_weights, W1, b1, W2, b2, gamma,
           beta):
    p_cls = _edge_layout(adj_pos[0])
    p_lit = _edge_layout(adj_pos[1])
    n_cls = _edge_layout(adj_neg[0])
    n_lit = _edge_layout(adj_neg[1])
    wb = jnp.broadcast_to(literals_weights[:, None], (4, 16))
    zinv = _z_kernel()(p_cls, p_lit, n_cls, n_lit, wb)
    zcol = zinv[:NL, 0:1]
    h = x
    nh = _tc_ln(x, gamma[0:1], beta[0:1])
    for i in range(NLAYER):
        nh_p = jnp.pad(nh, ((0, NLP - NL), (0, 0)))
        nh_c = nh_p.reshape(NLP, NCH, DC).transpose(1, 0, 2)
        o_c = _mp_kernel()(nh_c, p_cls, p_lit, n_cls, n_lit, wb)
        o = o_c.transpose(1, 0, 2).reshape(NLP, D)[:NL]
        h, nh = _tc_tail(h, o, zcol, gamma[2 * i + 1:2 * i + 2],
                         beta[2 * i + 1:2 * i + 2],
                         gamma[2 * i + 2:2 * i + 3],
                         beta[2 * i + 2:2 * i + 3],
                         W1[i], b1[i:i + 1], W2[i], b2[i:i + 1])
    return nh


# speedup vs baseline: 1.3661x; 1.0896x over previous
import functools

import jax
import jax.numpy as jnp
from jax import lax
from jax.experimental import pallas as pl
from jax.experimental.pallas import tpu as pltpu
from jax.experimental.pallas import tpu_sc as plsc

NL = 10000
NCL = 10000
NE = 160000
D = 256
DFF = 1024
NLAYER = 3

NSC = 2
NTEC = 16
NLP = 10240
EW = NE // NTEC
B = 256
EWP = 10240
NB = EWP // B
PADROW = NL + 16
DC = 32
NCH = D // DC
NR = NCH // NSC
RS = NLP // NTEC
SB = 128
NSB = RS // SB
EPS = 1e-6

@functools.cache
def _mesh():
    return plsc.VectorSubcoreMesh(
        core_axis_name="c", subcore_axis_name="s", num_cores=NSC,
        num_subcores=NTEC)


def _fill(ref, rows, cols, val):
    vec = jnp.full((16,), val, jnp.float32)

    def body(i, carry):
        for q in range(cols // 16):
            ref[i, pl.ds(q * 16, 16)] = vec
        return carry

    lax.fori_loop(0, rows, body, 0)


def _z_body(p_cls, p_lit, n_cls, n_lit, wb, zinv_out,
            zt_p, zt_n, zw, pcv, plv, ncv, nlv, gz, ga, gb, buf_a, buf_b,
            wvm, sem0, sem1, sem2, sem3):
    c = lax.axis_index("c")
    s = lax.axis_index("s")
    row0 = s * RS
    pltpu.sync_copy(p_cls.at[s], pcv)
    pltpu.sync_copy(p_lit.at[s], plv)
    pltpu.sync_copy(n_cls.at[s], ncv)
    pltpu.sync_copy(n_lit.at[s], nlv)
    pltpu.sync_copy(wb, wvm)
    _fill(buf_a, RS, 16, 0.0)
    pltpu.sync_copy(buf_a, zt_p.at[pl.ds(row0, RS)])
    pltpu.sync_copy(buf_a, zt_n.at[pl.ds(row0, RS)])
    pltpu.sync_copy(buf_a, zw.at[pl.ds(row0, RS)])
    _fill(gz, B, 16, 1.0)
    plsc.subcore_barrier()

    def deg(idx_v, acc_sp):
        def body(t, carry):
            d0 = pltpu.async_copy(gz, acc_sp.at[idx_v.at[2 * t]], sem0,
                                  add=True)
            d1 = pltpu.async_copy(gz, acc_sp.at[idx_v.at[2 * t + 1]], sem1,
                                  add=True)
            d0.wait()
            d1.wait()
            return carry

        lax.fori_loop(0, NB // 2, body, 0)

    deg(pcv, zt_p)
    deg(ncv, zt_n)
    plsc.subcore_barrier()
    pltpu.sync_copy(zt_p.at[pl.ds(row0, RS)], buf_a)
    pltpu.sync_copy(zt_n.at[pl.ds(row0, RS)], buf_b)
    w0 = wvm[0]
    w1 = wvm[1]
    w2 = wvm[2]
    w3 = wvm[3]

    def comb(i, carry):
        a = buf_a[i]
        b = buf_b[i]
        buf_a[i] = w0 * a + w1 * b
        buf_b[i] = w2 * a + w3 * b
        return carry

    lax.fori_loop(0, RS, comb, 0)
    pltpu.sync_copy(buf_a, zt_p.at[pl.ds(row0, RS)])
    pltpu.sync_copy(buf_b, zt_n.at[pl.ds(row0, RS)])
    plsc.subcore_barrier()

    def lit_scatter(cls_v, lit_v, src_sp):
        def body(t, carry):
            j0 = 2 * t
            j1 = 2 * t + 1
            dg0 = pltpu.async_copy(src_sp.at[cls_v.at[j0]], ga, sem0)
            dg1 = pltpu.async_copy(src_sp.at[cls_v.at[j1]], gb, sem1)
            dg0.wait()
            ds0 = pltpu.async_copy(ga, zw.at[lit_v.at[j0]], sem2, add=True)
            dg1.wait()
            ds1 = pltpu.async_copy(gb, zw.at[lit_v.at[j1]], sem3, add=True)
            ds0.wait()
            ds1.wait()
            return carry

        lax.fori_loop(0, NB // 2, body, 0)

    lit_scatter(pcv, plv, zt_p)
    lit_scatter(ncv, nlv, zt_n)
    plsc.subcore_barrier()
    pltpu.sync_copy(zw.at[pl.ds(row0, RS)], buf_a)

    def recip(i, carry):
        buf_a[i] = 1.0 / (buf_a[i] + EPS)
        return carry

    lax.fori_loop(0, RS, recip, 0)

    @pl.when(c == 0)
    def _():
        pltpu.sync_copy(buf_a, zinv_out.at[pl.ds(row0, RS)])


@functools.cache
def _z_kernel():
    return pl.kernel(
        _z_body,
        out_type=jax.ShapeDtypeStruct((NLP, 16), jnp.float32),
        mesh=_mesh(),
        compiler_params=pltpu.CompilerParams(use_tc_tiling_on_sc=False),
        scratch_types=[
            pltpu.VMEM_SHARED((NLP, 16), jnp.float32),
            pltpu.VMEM_SHARED((NLP, 16), jnp.float32),
            pltpu.VMEM_SHARED((NLP, 16), jnp.float32),
            pltpu.VMEM((NB, B), jnp.int32),
            pltpu.VMEM((NB, B), jnp.int32),
            pltpu.VMEM((NB, B), jnp.int32),
            pltpu.VMEM((NB, B), jnp.int32),
            pltpu.VMEM((B, 16), jnp.float32),
            pltpu.VMEM((B, 16), jnp.float32),
            pltpu.VMEM((B, 16), jnp.float32),
            pltpu.VMEM((RS, 16), jnp.float32),
            pltpu.VMEM((RS, 16), jnp.float32),
            pltpu.VMEM((4, 16), jnp.float32),
            pltpu.SemaphoreType.DMA,
            pltpu.SemaphoreType.DMA,
            pltpu.SemaphoreType.DMA,
            pltpu.SemaphoreType.DMA,
        ],
    )


def _mp_body(nh, p_cls, p_lit, n_cls, n_lit, wb, o_out,
             wv_sp, tp_sp, tn_sp, pcv, plv, ncv, nlv, ga, gb, buf_a, buf_b,
             wvm, sem0, sem1, sem2, sem3):
    c = lax.axis_index("c")
    s = lax.axis_index("s")
    row0 = s * RS
    pltpu.sync_copy(p_cls.at[s], pcv)
    pltpu.sync_copy(p_lit.at[s], plv)
    pltpu.sync_copy(n_cls.at[s], ncv)
    pltpu.sync_copy(n_lit.at[s], nlv)
    pltpu.sync_copy(wb, wvm)
    w0 = wvm[0]
    w1 = wvm[1]
    w2 = wvm[2]
    w3 = wvm[3]
    def edge_pass(gat_v, sct_v, src, dst_sp):
        def body(t, carry):
            j0 = 2 * t
            j1 = 2 * t + 1
            dg0 = pltpu.async_copy(src.at[gat_v.at[j0]], ga, sem0)
            dg1 = pltpu.async_copy(src.at[gat_v.at[j1]], gb, sem1)
            dg0.wait()
            ds0 = pltpu.async_copy(ga, dst_sp.at[sct_v.at[j0]], sem2,
                                   add=True)
            dg1.wait()
            ds1 = pltpu.async_copy(gb, dst_sp.at[sct_v.at[j1]], sem3,
                                   add=True)
            ds0.wait()
            ds1.wait()
            return carry

        lax.fori_loop(0, NB // 2, body, 0)

    for r in range(NR):
        ch = NSC * r + c
        pltpu.sync_copy(nh.at[pl.ds(row0, RS), pl.ds(ch * DC, DC)],
                        wv_sp.at[pl.ds(row0, RS)])
        _fill(buf_a, SB, DC, 0.0)
        for k in range(NSB):
            pltpu.sync_copy(buf_a, tp_sp.at[pl.ds(row0 + k * SB, SB)])
            pltpu.sync_copy(buf_a, tn_sp.at[pl.ds(row0 + k * SB, SB)])
        plsc.subcore_barrier()

        edge_pass(plv, pcv, wv_sp, tp_sp)
        edge_pass(nlv, ncv, wv_sp, tn_sp)
        plsc.subcore_barrier()
        for k in range(NSB):
            base = row0 + k * SB
            pltpu.sync_copy(tp_sp.at[pl.ds(base, SB)], buf_a)
            pltpu.sync_copy(tn_sp.at[pl.ds(base, SB)], buf_b)

            def comb(i, carry):
                for q in range(DC // 16):
                    sl = pl.ds(q * 16, 16)
                    a = buf_a[i, sl]
                    b = buf_b[i, sl]
                    buf_a[i, sl] = w0 * a + w1 * b
                    buf_b[i, sl] = w2 * a + w3 * b
                return carry

            lax.fori_loop(0, SB, comb, 0)
            pltpu.sync_copy(buf_a, tp_sp.at[pl.ds(base, SB)])
            pltpu.sync_copy(buf_b, tn_sp.at[pl.ds(base, SB)])
        _fill(buf_a, SB, DC, 0.0)
        for k in range(NSB):
            pltpu.sync_copy(buf_a, wv_sp.at[pl.ds(row0 + k * SB, SB)])
        plsc.subcore_barrier()
        edge_pass(pcv, plv, tp_sp, wv_sp)
        edge_pass(ncv, nlv, tn_sp, wv_sp)
        plsc.subcore_barrier()
        pltpu.sync_copy(wv_sp.at[pl.ds(row0, RS)],
                        o_out.at[pl.ds(row0, RS), pl.ds(ch * DC, DC)])


@functools.cache
def _mp_kernel():
    return pl.kernel(
        _mp_body,
        out_type=jax.ShapeDtypeStruct((NLP, D), jnp.float32),
        mesh=_mesh(),
        compiler_params=pltpu.CompilerParams(use_tc_tiling_on_sc=False),
        scratch_types=[
            pltpu.VMEM_SHARED((NLP, DC), jnp.float32),
            pltpu.VMEM_SHARED((NLP, DC), jnp.float32),
            pltpu.VMEM_SHARED((NLP, DC), jnp.float32),
            pltpu.VMEM((NB, B), jnp.int32),
            pltpu.VMEM((NB, B), jnp.int32),
            pltpu.VMEM((NB, B), jnp.int32),
            pltpu.VMEM((NB, B), jnp.int32),
            pltpu.VMEM((B, DC), jnp.float32),
            pltpu.VMEM((B, DC), jnp.float32),
            pltpu.VMEM((SB, DC), jnp.float32),
            pltpu.VMEM((SB, DC), jnp.float32),
            pltpu.VMEM((4, 16), jnp.float32),
            pltpu.SemaphoreType.DMA,
            pltpu.SemaphoreType.DMA,
            pltpu.SemaphoreType.DMA,
            pltpu.SemaphoreType.DMA,
        ],
    )


def _ln_block(x_ref, g_ref, b_ref, o_ref):
    x = x_ref[...]
    mu = jnp.mean(x, axis=-1, keepdims=True)
    var = jnp.mean((x - mu) ** 2, axis=-1, keepdims=True)
    o_ref[...] = (x - mu) / jnp.sqrt(var + EPS) * g_ref[...] + b_ref[...]


_TC_R = 1024


def _tc_ln(x, g, b):
    return pl.pallas_call(
        _ln_block,
        grid=(NLP // _TC_R,),
        in_specs=[
            pl.BlockSpec((_TC_R, D), lambda i: (i, 0)),
            pl.BlockSpec((1, D), lambda i: (0, 0)),
            pl.BlockSpec((1, D), lambda i: (0, 0)),
        ],
        out_specs=pl.BlockSpec((_TC_R, D), lambda i: (i, 0)),
        out_shape=jax.ShapeDtypeStruct((NLP, D), jnp.float32),
    )(x, g, b)


def _tail_block(h_ref, o_ref, zi_ref, ga_ref, ba_ref, gb_ref, bb_ref,
                w1_ref, b1_ref, w2_ref, b2_ref, hn_ref, ln_ref):
    h2 = h_ref[...] + o_ref[...] * zi_ref[...]
    mu = jnp.mean(h2, axis=-1, keepdims=True)
    var = jnp.mean((h2 - mu) ** 2, axis=-1, keepdims=True)
    n2 = (h2 - mu) / jnp.sqrt(var + EPS) * ga_ref[...] + ba_ref[...]
    ff = jnp.maximum(jnp.dot(n2, w1_ref[...]) + b1_ref[...], 0.0)
    h3 = h2 + jnp.dot(ff, w2_ref[...]) + b2_ref[...]
    hn_ref[...] = h3
    mu2 = jnp.mean(h3, axis=-1, keepdims=True)
    var2 = jnp.mean((h3 - mu2) ** 2, axis=-1, keepdims=True)
    ln_ref[...] = (h3 - mu2) / jnp.sqrt(var2 + EPS) * gb_ref[...] + bb_ref[...]


def _tc_tail(h, o, zi, ga, ba, gb, bb, w1, b1, w2, b2):
    row = pl.BlockSpec((_TC_R, D), lambda i: (i, 0))
    vec = pl.BlockSpec((1, D), lambda i: (0, 0))
    return pl.pallas_call(
        _tail_block,
        grid=(NLP // _TC_R,),
        in_specs=[
            row, row,
            pl.BlockSpec((_TC_R, 1), lambda i: (i, 0)),
            vec, vec, vec, vec,
            pl.BlockSpec((D, DFF), lambda i: (0, 0)),
            pl.BlockSpec((1, DFF), lambda i: (0, 0)),
            pl.BlockSpec((DFF, D), lambda i: (0, 0)),
            vec,
        ],
        out_specs=[row, row],
        out_shape=[
            jax.ShapeDtypeStruct((NLP, D), jnp.float32),
            jax.ShapeDtypeStruct((NLP, D), jnp.float32),
        ],
    )(h, o, zi, ga, ba, gb, bb, w1, b1, w2, b2)


def _edge_layout(idx_row):
    per_tec = idx_row.reshape(NTEC, EW)
    padded = jnp.pad(per_tec, ((0, 0), (0, EWP - EW)),
                     constant_values=PADROW)
    return padded.reshape(NTEC, NB, B)


def kernel(x, adj_pos, adj_neg, literals_weights, W1, b1, W2, b2, gamma,
           beta):
    p_cls = _edge_layout(adj_pos[0])
    p_lit = _edge_layout(adj_pos[1])
    n_cls = _edge_layout(adj_neg[0])
    n_lit = _edge_layout(adj_neg[1])
    wb = jnp.broadcast_to(literals_weights[:, None], (4, 16))
    zinv = _z_kernel()(p_cls, p_lit, n_cls, n_lit, wb)
    zcol = zinv[:, 0:1]
    h = jnp.pad(x, ((0, NLP - NL), (0, 0)))
    nh = _tc_ln(h, gamma[0:1], beta[0:1])
    for i in range(NLAYER):
        o = _mp_kernel()(nh, p_cls, p_lit, n_cls, n_lit, wb)
        h, nh = _tc_tail(h, o, zcol, gamma[2 * i + 1:2 * i + 2],
                         beta[2 * i + 1:2 * i + 2],
                         gamma[2 * i + 2:2 * i + 3],
                         beta[2 * i + 2:2 * i + 3],
                         W1[i], b1[i:i + 1], W2[i], b2[i:i + 1])
    return nh[:NL]
